# baseline (device time: 698400 ns/iter reference)
import jax
import jax.numpy as jnp
from jax import lax
from jax.experimental import pallas as pl
from jax.experimental.pallas import tpu as pltpu

N_DEV = 4


def _push_allgather(arr, collective_id, scatter):
    chunk_shape = arr.shape[1:] if scatter else arr.shape

    def body(a_ref, out_ref, send_sems, recv_sems, local_sem):
        my = lax.axis_index("i")

        barrier = pltpu.get_barrier_semaphore()
        for d in range(1, N_DEV):
            pl.semaphore_signal(
                barrier, inc=1,
                device_id=((my + d) % N_DEV,),
                device_id_type=pl.DeviceIdType.MESH,
            )
        pl.semaphore_wait(barrier, N_DEV - 1)

        src_local = a_ref.at[my] if scatter else a_ref
        cp = pltpu.make_async_copy(src_local, out_ref.at[my], local_sem)
        cp.start()

        sends = []
        for d in range(1, N_DEV):
            peer = (my + d) % N_DEV
            rdma = pltpu.make_async_remote_copy(
                src_ref=a_ref.at[peer] if scatter else a_ref,
                dst_ref=out_ref.at[my],
                send_sem=send_sems.at[d - 1],
                recv_sem=recv_sems.at[d - 1],
                device_id=(peer,),
                device_id_type=pl.DeviceIdType.MESH,
            )
            rdma.start()
            sends.append(rdma)

        cp.wait()
        for r in sends:
            r.wait_send()

        for d in range(1, N_DEV):
            src_pos = (my - d) % N_DEV
            recv = pltpu.make_async_remote_copy(
                src_ref=a_ref.at[src_pos] if scatter else a_ref,
                dst_ref=out_ref.at[src_pos],
                send_sem=send_sems.at[0],
                recv_sem=recv_sems.at[d - 1],
                device_id=(src_pos,),
                device_id_type=pl.DeviceIdType.MESH,
            )
            recv.wait_recv()

    return pl.pallas_call(
        body,
        out_shape=jax.ShapeDtypeStruct((N_DEV, *chunk_shape), arr.dtype),
        in_specs=[pl.BlockSpec(memory_space=pl.ANY)],
        out_specs=pl.BlockSpec(memory_space=pl.ANY),
        scratch_shapes=[
            pltpu.SemaphoreType.DMA((N_DEV - 1,)),
            pltpu.SemaphoreType.DMA((N_DEV - 1,)),
            pltpu.SemaphoreType.DMA,
        ],
        compiler_params=pltpu.CompilerParams(collective_id=collective_id),
    )(arr)


def _ring_hop(send_cw, send_ccw, collective_id):

    def body(cw_ref, ccw_ref, rcw_ref, rccw_ref, send_sems, recv_sems):
        my = lax.axis_index("i")
        right = (my + 1) % N_DEV
        left = (my - 1) % N_DEV

        barrier = pltpu.get_barrier_semaphore()
        for nbr in (left, right):
            pl.semaphore_signal(
                barrier, inc=1,
                device_id=(nbr,), device_id_type=pl.DeviceIdType.MESH,
            )
        pl.semaphore_wait(barrier, 2)

        cw = pltpu.make_async_remote_copy(
            src_ref=cw_ref, dst_ref=rcw_ref,
            send_sem=send_sems.at[0], recv_sem=recv_sems.at[0],
            device_id=(right,), device_id_type=pl.DeviceIdType.MESH,
        )
        ccw = pltpu.make_async_remote_copy(
            src_ref=ccw_ref, dst_ref=rccw_ref,
            send_sem=send_sems.at[1], recv_sem=recv_sems.at[1],
            device_id=(left,), device_id_type=pl.DeviceIdType.MESH,
        )
        cw.start()
        ccw.start()
        cw.wait_send()
        ccw.wait_send()
        cw.wait_recv()
        ccw.wait_recv()

    shape = jax.ShapeDtypeStruct(send_cw.shape, send_cw.dtype)
    return pl.pallas_call(
        body,
        out_shape=(shape, shape),
        in_specs=[
            pl.BlockSpec(memory_space=pl.ANY),
            pl.BlockSpec(memory_space=pl.ANY),
        ],
        out_specs=(
            pl.BlockSpec(memory_space=pl.ANY),
            pl.BlockSpec(memory_space=pl.ANY),
        ),
        scratch_shapes=[
            pltpu.SemaphoreType.DMA((2,)),
            pltpu.SemaphoreType.DMA((2,)),
        ],
        compiler_params=pltpu.CompilerParams(collective_id=collective_id),
    )(send_cw, send_ccw)


def _ring_ag(q, collective_id):
    m_chunk, n = q.shape
    half_m = m_chunk // 2
    n_hops = N_DEV - 1

    def body(q_ref, out_ref, cw_buf, ccw_buf,
             send_cw, send_ccw, recv_cw, recv_ccw, cp_sems):
        my = lax.axis_index("i")
        right = (my + 1) % N_DEV
        left = (my - 1) % N_DEV

        barrier = pltpu.get_barrier_semaphore()
        for nbr in (left, right):
            pl.semaphore_signal(
                barrier, inc=1,
                device_id=(nbr,), device_id_type=pl.DeviceIdType.MESH,
            )
        pl.semaphore_wait(barrier, 2)

        own = pltpu.make_async_copy(
            q_ref, out_ref.at[pl.ds(my * m_chunk, m_chunk), :], cp_sems.at[0]
        )
        own.start()

        sends = []
        for h in range(n_hops):
            cw = pltpu.make_async_remote_copy(
                src_ref=q_ref.at[0:half_m, :] if h == 0 else cw_buf.at[h - 1],
                dst_ref=cw_buf.at[h],
                send_sem=send_cw.at[h], recv_sem=recv_cw.at[h],
                device_id=(right,), device_id_type=pl.DeviceIdType.MESH,
            )
            ccw = pltpu.make_async_remote_copy(
                src_ref=q_ref.at[half_m:m_chunk, :] if h == 0
                else ccw_buf.at[h - 1],
                dst_ref=ccw_buf.at[h],
                send_sem=send_ccw.at[h], recv_sem=recv_ccw.at[h],
                device_id=(left,), device_id_type=pl.DeviceIdType.MESH,
            )
            cw.start()
            ccw.start()
            sends += [cw, ccw]
            cw.wait_recv()
            ccw.wait_recv()
            org_cw = (my - 1 - h) % N_DEV
            org_ccw = (my + 1 + h) % N_DEV
            st_cw = pltpu.make_async_copy(
                cw_buf.at[h],
                out_ref.at[pl.ds(org_cw * m_chunk, half_m), :],
                cp_sems.at[1 + 2 * h],
            )
            st_ccw = pltpu.make_async_copy(
                ccw_buf.at[h],
                out_ref.at[pl.ds(org_ccw * m_chunk + half_m, half_m), :],
                cp_sems.at[2 + 2 * h],
            )
            st_cw.start()
            st_ccw.start()

        for r in sends:
            r.wait_send()
        own.wait()
        for h in range(n_hops):
            pltpu.make_async_copy(
                cw_buf.at[h],
                out_ref.at[pl.ds(0, half_m), :],
                cp_sems.at[1 + 2 * h],
            ).wait()
            pltpu.make_async_copy(
                ccw_buf.at[h],
                out_ref.at[pl.ds(0, half_m), :],
                cp_sems.at[2 + 2 * h],
            ).wait()

    return pl.pallas_call(
        body,
        out_shape=jax.ShapeDtypeStruct((N_DEV * m_chunk, n), q.dtype),
        in_specs=[pl.BlockSpec(memory_space=pl.ANY)],
        out_specs=pl.BlockSpec(memory_space=pl.ANY),
        scratch_shapes=[
            pltpu.VMEM((n_hops, half_m, n), q.dtype),
            pltpu.VMEM((n_hops, half_m, n), q.dtype),
            pltpu.SemaphoreType.DMA((n_hops,)),
            pltpu.SemaphoreType.DMA((n_hops,)),
            pltpu.SemaphoreType.DMA((n_hops,)),
            pltpu.SemaphoreType.DMA((n_hops,)),
            pltpu.SemaphoreType.DMA((1 + 2 * n_hops,)),
        ],
        compiler_params=pltpu.CompilerParams(collective_id=collective_id),
    )(q)


def _ring_ag_dequant(q, scale, collective_id):
    m_chunk, n = q.shape
    half_m = m_chunk // 2
    n_hops = N_DEV - 1
    t_rows = 128

    def body(q_ref, scale_ref, out_ref, cw_buf, ccw_buf, qt, yv,
             send_cw, send_ccw, recv_cw, recv_ccw, qt_sems, out_sems):
        my = lax.axis_index("i")
        right = (my + 1) % N_DEV
        left = (my - 1) % N_DEV

        barrier = pltpu.get_barrier_semaphore()
        for nbr in (left, right):
            pl.semaphore_signal(
                barrier, inc=1,
                device_id=(nbr,), device_id_type=pl.DeviceIdType.MESH,
            )
        pl.semaphore_wait(barrier, 2)

        s = scale_ref[0]
        out_dmas = [None, None]

        def emit_tile(value, out_row, slot):
            if out_dmas[slot] is not None:
                out_dmas[slot].wait()
            yv[slot] = value
            d = pltpu.make_async_copy(
                yv.at[slot],
                out_ref.at[pl.ds(out_row, t_rows), :],
                out_sems.at[slot],
            )
            d.start()
            out_dmas[slot] = d

        def hop_rdmas(h):
            cw = pltpu.make_async_remote_copy(
                src_ref=q_ref.at[0:half_m, :] if h == 0 else cw_buf.at[h - 1],
                dst_ref=cw_buf.at[h],
                send_sem=send_cw.at[h], recv_sem=recv_cw.at[h],
                device_id=(right,), device_id_type=pl.DeviceIdType.MESH,
            )
            ccw = pltpu.make_async_remote_copy(
                src_ref=q_ref.at[half_m:m_chunk, :] if h == 0
                else ccw_buf.at[h - 1],
                dst_ref=ccw_buf.at[h],
                send_sem=send_ccw.at[h], recv_sem=recv_ccw.at[h],
                device_id=(left,), device_id_type=pl.DeviceIdType.MESH,
            )
            cw.start()
            ccw.start()
            return cw, ccw

        cw, ccw = hop_rdmas(0)
        sends = [cw, ccw]

        n_own = m_chunk // t_rows
        stage = [None, None]
        for t in range(n_own):
            slot = t % 2
            cp = pltpu.make_async_copy(
                q_ref.at[pl.ds(t * t_rows, t_rows), :], qt.at[slot],
                qt_sems.at[slot],
            )
            cp.start()
            stage[slot] = cp
            if t > 0:
                prev = (t - 1) % 2
                stage[prev].wait()
                emit_tile(
                    qt[prev].astype(jnp.float32) * s,
                    my * m_chunk + (t - 1) * t_rows,
                    prev,
                )
        last = (n_own - 1) % 2
        stage[last].wait()
        emit_tile(
            qt[last].astype(jnp.float32) * s,
            my * m_chunk + (n_own - 1) * t_rows,
            last,
        )

        for h in range(n_hops):
            cw.wait_recv()
            ccw.wait_recv()
            if h + 1 < n_hops:
                cw, ccw = hop_rdmas(h + 1)
                sends += [cw, ccw]
            org_cw = ((my - 1 - h) % N_DEV) * m_chunk
            org_ccw = ((my + 1 + h) % N_DEV) * m_chunk + half_m
            for t in range(half_m // t_rows):
                emit_tile(
                    cw_buf[h, pl.ds(t * t_rows, t_rows), :].astype(
                        jnp.float32) * s,
                    org_cw + t * t_rows,
                    t % 2,
                )
                emit_tile(
                    ccw_buf[h, pl.ds(t * t_rows, t_rows), :].astype(
                        jnp.float32) * s,
                    org_ccw + t * t_rows,
                    (t + 1) % 2,
                )

        for r in sends:
            r.wait_send()
        for d in out_dmas:
            d.wait()

    return pl.pallas_call(
        body,
        out_shape=jax.ShapeDtypeStruct((N_DEV * m_chunk, n), jnp.float32),
        in_specs=[
            pl.BlockSpec(memory_space=pl.ANY),
            pl.BlockSpec(memory_space=pltpu.MemorySpace.SMEM),
        ],
        out_specs=pl.BlockSpec(memory_space=pl.ANY),
        scratch_shapes=[
            pltpu.VMEM((n_hops, half_m, n), q.dtype),
            pltpu.VMEM((n_hops, half_m, n), q.dtype),
            pltpu.VMEM((2, t_rows, n), q.dtype),
            pltpu.VMEM((2, t_rows, n), jnp.float32),
            pltpu.SemaphoreType.DMA((n_hops,)),
            pltpu.SemaphoreType.DMA((n_hops,)),
            pltpu.SemaphoreType.DMA((n_hops,)),
            pltpu.SemaphoreType.DMA((n_hops,)),
            pltpu.SemaphoreType.DMA((2,)),
            pltpu.SemaphoreType.DMA((2,)),
        ],
        compiler_params=pltpu.CompilerParams(
            collective_id=collective_id,
            vmem_limit_bytes=64 * 1024 * 1024,
        ),
    )(q, scale)


def kernel(x, w_mat):
    m = x.shape[0]
    n = w_mat.shape[1]
    m_chunk = m // N_DEV
    half = n // 2

    my = lax.axis_index("i")
    partial = jnp.dot(
        x.astype(jnp.bfloat16),
        w_mat.astype(jnp.bfloat16),
        preferred_element_type=jnp.bfloat16,
    )

    half_m = m_chunk // 2
    p16 = partial.reshape(2 * N_DEV, half_m, n)

    def hchunk(i):
        return lax.dynamic_index_in_dim(
            p16, i % (2 * N_DEV), axis=0, keepdims=False
        )

    acc_cw = hchunk(2 * ((my - 1) % N_DEV))
    acc_ccw = hchunk(2 * ((my + 1) % N_DEV) + 1)
    for h in range(N_DEV - 1):
        r_cw, r_ccw = _ring_hop(acc_cw, acc_ccw, collective_id=h)
        acc_cw = r_cw + hchunk(2 * ((my - 2 - h) % N_DEV))
        acc_ccw = r_ccw + hchunk(2 * ((my + 2 + h) % N_DEV) + 1)
    my_chunk = jnp.concatenate([acc_cw, acc_ccw], axis=0).astype(jnp.float32)
    my_chunk = jnp.maximum(my_chunk, 0.0)

    amax_tile = jnp.full((8, 128), jnp.max(my_chunk), jnp.float32)
    amaxes = _push_allgather(amax_tile, collective_id=3, scatter=False)
    scale = jnp.max(amaxes) / 448.0

    q = (my_chunk / scale).astype(jnp.float8_e4m3fn)
    return _ring_ag_dequant(q, jnp.reshape(scale, (1,)), collective_id=4)


# device time: 652262 ns/iter; 1.0707x vs baseline; 1.0707x over previous
import jax
import jax.numpy as jnp
from jax import lax
from jax.experimental import pallas as pl
from jax.experimental.pallas import tpu as pltpu

N_DEV = 4


def _push_allgather(arr, collective_id, scatter):
    chunk_shape = arr.shape[1:] if scatter else arr.shape

    def body(a_ref, out_ref, send_sems, recv_sems, local_sem):
        my = lax.axis_index("i")

        barrier = pltpu.get_barrier_semaphore()
        for d in range(1, N_DEV):
            pl.semaphore_signal(
                barrier, inc=1,
                device_id=((my + d) % N_DEV,),
                device_id_type=pl.DeviceIdType.MESH,
            )
        pl.semaphore_wait(barrier, N_DEV - 1)

        src_local = a_ref.at[my] if scatter else a_ref
        cp = pltpu.make_async_copy(src_local, out_ref.at[my], local_sem)
        cp.start()

        sends = []
        for d in range(1, N_DEV):
            peer = (my + d) % N_DEV
            rdma = pltpu.make_async_remote_copy(
                src_ref=a_ref.at[peer] if scatter else a_ref,
                dst_ref=out_ref.at[my],
                send_sem=send_sems.at[d - 1],
                recv_sem=recv_sems.at[d - 1],
                device_id=(peer,),
                device_id_type=pl.DeviceIdType.MESH,
            )
            rdma.start()
            sends.append(rdma)

        cp.wait()
        for r in sends:
            r.wait_send()

        for d in range(1, N_DEV):
            src_pos = (my - d) % N_DEV
            recv = pltpu.make_async_remote_copy(
                src_ref=a_ref.at[src_pos] if scatter else a_ref,
                dst_ref=out_ref.at[src_pos],
                send_sem=send_sems.at[0],
                recv_sem=recv_sems.at[d - 1],
                device_id=(src_pos,),
                device_id_type=pl.DeviceIdType.MESH,
            )
            recv.wait_recv()

    return pl.pallas_call(
        body,
        out_shape=jax.ShapeDtypeStruct((N_DEV, *chunk_shape), arr.dtype),
        in_specs=[pl.BlockSpec(memory_space=pl.ANY)],
        out_specs=pl.BlockSpec(memory_space=pl.ANY),
        scratch_shapes=[
            pltpu.SemaphoreType.DMA((N_DEV - 1,)),
            pltpu.SemaphoreType.DMA((N_DEV - 1,)),
            pltpu.SemaphoreType.DMA,
        ],
        compiler_params=pltpu.CompilerParams(collective_id=collective_id),
    )(arr)


def _ring_hop(send_cw, send_ccw, collective_id):

    def body(cw_ref, ccw_ref, rcw_ref, rccw_ref, send_sems, recv_sems):
        my = lax.axis_index("i")
        right = (my + 1) % N_DEV
        left = (my - 1) % N_DEV

        barrier = pltpu.get_barrier_semaphore()
        for nbr in (left, right):
            pl.semaphore_signal(
                barrier, inc=1,
                device_id=(nbr,), device_id_type=pl.DeviceIdType.MESH,
            )
        pl.semaphore_wait(barrier, 2)

        cw = pltpu.make_async_remote_copy(
            src_ref=cw_ref, dst_ref=rcw_ref,
            send_sem=send_sems.at[0], recv_sem=recv_sems.at[0],
            device_id=(right,), device_id_type=pl.DeviceIdType.MESH,
        )
        ccw = pltpu.make_async_remote_copy(
            src_ref=ccw_ref, dst_ref=rccw_ref,
            send_sem=send_sems.at[1], recv_sem=recv_sems.at[1],
            device_id=(left,), device_id_type=pl.DeviceIdType.MESH,
        )
        cw.start()
        ccw.start()
        cw.wait_send()
        ccw.wait_send()
        cw.wait_recv()
        ccw.wait_recv()

    shape = jax.ShapeDtypeStruct(send_cw.shape, send_cw.dtype)
    return pl.pallas_call(
        body,
        out_shape=(shape, shape),
        in_specs=[
            pl.BlockSpec(memory_space=pl.ANY),
            pl.BlockSpec(memory_space=pl.ANY),
        ],
        out_specs=(
            pl.BlockSpec(memory_space=pl.ANY),
            pl.BlockSpec(memory_space=pl.ANY),
        ),
        scratch_shapes=[
            pltpu.SemaphoreType.DMA((2,)),
            pltpu.SemaphoreType.DMA((2,)),
        ],
        compiler_params=pltpu.CompilerParams(collective_id=collective_id),
    )(send_cw, send_ccw)


def _ring_ag(q, collective_id):
    m_chunk, n = q.shape
    half_m = m_chunk // 2
    n_hops = N_DEV - 1

    def body(q_ref, out_ref, cw_buf, ccw_buf,
             send_cw, send_ccw, recv_cw, recv_ccw, cp_sems):
        my = lax.axis_index("i")
        right = (my + 1) % N_DEV
        left = (my - 1) % N_DEV

        barrier = pltpu.get_barrier_semaphore()
        for nbr in (left, right):
            pl.semaphore_signal(
                barrier, inc=1,
                device_id=(nbr,), device_id_type=pl.DeviceIdType.MESH,
            )
        pl.semaphore_wait(barrier, 2)

        own = pltpu.make_async_copy(
            q_ref, out_ref.at[pl.ds(my * m_chunk, m_chunk), :], cp_sems.at[0]
        )
        own.start()

        sends = []
        for h in range(n_hops):
            cw = pltpu.make_async_remote_copy(
                src_ref=q_ref.at[0:half_m, :] if h == 0 else cw_buf.at[h - 1],
                dst_ref=cw_buf.at[h],
                send_sem=send_cw.at[h], recv_sem=recv_cw.at[h],
                device_id=(right,), device_id_type=pl.DeviceIdType.MESH,
            )
            ccw = pltpu.make_async_remote_copy(
                src_ref=q_ref.at[half_m:m_chunk, :] if h == 0
                else ccw_buf.at[h - 1],
                dst_ref=ccw_buf.at[h],
                send_sem=send_ccw.at[h], recv_sem=recv_ccw.at[h],
                device_id=(left,), device_id_type=pl.DeviceIdType.MESH,
            )
            cw.start()
            ccw.start()
            sends += [cw, ccw]
            cw.wait_recv()
            ccw.wait_recv()
            org_cw = (my - 1 - h) % N_DEV
            org_ccw = (my + 1 + h) % N_DEV
            st_cw = pltpu.make_async_copy(
                cw_buf.at[h],
                out_ref.at[pl.ds(org_cw * m_chunk, half_m), :],
                cp_sems.at[1 + 2 * h],
            )
            st_ccw = pltpu.make_async_copy(
                ccw_buf.at[h],
                out_ref.at[pl.ds(org_ccw * m_chunk + half_m, half_m), :],
                cp_sems.at[2 + 2 * h],
            )
            st_cw.start()
            st_ccw.start()

        for r in sends:
            r.wait_send()
        own.wait()
        for h in range(n_hops):
            pltpu.make_async_copy(
                cw_buf.at[h],
                out_ref.at[pl.ds(0, half_m), :],
                cp_sems.at[1 + 2 * h],
            ).wait()
            pltpu.make_async_copy(
                ccw_buf.at[h],
                out_ref.at[pl.ds(0, half_m), :],
                cp_sems.at[2 + 2 * h],
            ).wait()

    return pl.pallas_call(
        body,
        out_shape=jax.ShapeDtypeStruct((N_DEV * m_chunk, n), q.dtype),
        in_specs=[pl.BlockSpec(memory_space=pl.ANY)],
        out_specs=pl.BlockSpec(memory_space=pl.ANY),
        scratch_shapes=[
            pltpu.VMEM((n_hops, half_m, n), q.dtype),
            pltpu.VMEM((n_hops, half_m, n), q.dtype),
            pltpu.SemaphoreType.DMA((n_hops,)),
            pltpu.SemaphoreType.DMA((n_hops,)),
            pltpu.SemaphoreType.DMA((n_hops,)),
            pltpu.SemaphoreType.DMA((n_hops,)),
            pltpu.SemaphoreType.DMA((1 + 2 * n_hops,)),
        ],
        compiler_params=pltpu.CompilerParams(collective_id=collective_id),
    )(q)


def kernel(x, w_mat):
    m = x.shape[0]
    n = w_mat.shape[1]
    m_chunk = m // N_DEV
    half = n // 2

    my = lax.axis_index("i")
    partial = jnp.dot(
        x.astype(jnp.bfloat16),
        w_mat.astype(jnp.bfloat16),
        preferred_element_type=jnp.bfloat16,
    )

    half_m = m_chunk // 2
    p16 = partial.reshape(2 * N_DEV, half_m, n)

    def hchunk(i):
        return lax.dynamic_index_in_dim(
            p16, i % (2 * N_DEV), axis=0, keepdims=False
        )

    acc_cw = hchunk(2 * ((my - 1) % N_DEV))
    acc_ccw = hchunk(2 * ((my + 1) % N_DEV) + 1)
    for h in range(N_DEV - 1):
        r_cw, r_ccw = _ring_hop(acc_cw, acc_ccw, collective_id=h)
        acc_cw = r_cw + hchunk(2 * ((my - 2 - h) % N_DEV))
        acc_ccw = r_ccw + hchunk(2 * ((my + 2 + h) % N_DEV) + 1)
    my_chunk = jnp.concatenate([acc_cw, acc_ccw], axis=0).astype(jnp.float32)
    my_chunk = jnp.maximum(my_chunk, 0.0)

    amax_tile = jnp.full((8, 128), jnp.max(my_chunk), jnp.float32)
    amaxes = _push_allgather(amax_tile, collective_id=3, scatter=False)
    scale = jnp.max(amaxes) / 448.0

    q = (my_chunk / scale).astype(jnp.float8_e4m3fn)
    q_full = _ring_ag(q, collective_id=4)
    return q_full.astype(jnp.float32) * scale
